# baseline (speedup 1.0000x reference)
import jax
import jax.numpy as jnp
from jax import lax
from jax.experimental import pallas as pl
from jax.experimental.pallas import tpu as pltpu
from jax.experimental.pallas import tpu_sc as plsc

N = 100000
D = 256
G = 512
OUT = 256
NC = 2
NS = 16
NW = NC * NS
NGRP = N // 8
GBASE = NGRP // NW
NBIG = NGRP - GBASE * NW
GCH = 24
NCH = 17
BPAD = 3152
MAXRUN = 544
NSL = D // 16

_LANE = None


def _sload(ref, i):
    return ref[pl.ds(i, 16)][0]


def _sstore(ref, i, val):
    lane = lax.broadcasted_iota(jnp.int32, (16,), 0)
    idx = jnp.zeros((16,), jnp.int32) + i
    x = jnp.zeros((16,), ref.dtype) + val
    plsc.store_scatter(ref, [idx], x, mask=lane == 0)


def _sc_body(x_hbm, b2_hbm, dstats, pstats,
             batch_v, xb0, xb1, acc, rlo, rhi, rseg, sem0, sem1):
    cid = lax.axis_index("c")
    sid = lax.axis_index("s")
    wid = sid * NC + cid
    gstart = wid * GBASE + jnp.minimum(wid, NBIG)
    groups_w = GBASE + (wid < NBIG).astype(jnp.int32)
    rows_w = groups_w * 8

    pltpu.sync_copy(b2_hbm.at[pl.ds(gstart * 8, BPAD)], batch_v)

    first = _sload(batch_v, 0)
    _sstore(rlo, 0, jnp.int32(0))
    _sstore(rseg, 0, first)

    def phase_a(i, j):
        r0 = i * 16
        a = batch_v[pl.ds(r0, 16)]
        b = batch_v[pl.ds(r0 + 1, 16)]
        pos = lax.broadcasted_iota(jnp.int32, (16,), 0) + (r0 + 1)
        m = (a != b) & (pos < rows_w)
        nb = plsc.all_reduce_population_count(m)[0]

        @pl.when(nb > 0)
        def _():
            plsc.store_compressed(rhi.at[pl.ds(j, 16)], pos, mask=m)
            plsc.store_compressed(rlo.at[pl.ds(j + 1, 16)], pos, mask=m)
            plsc.store_compressed(rseg.at[pl.ds(j + 1, 16)], b, mask=m)

        return j + nb

    jn = lax.fori_loop(0, (8 * (GBASE + 1) + 14) // 16, phase_a, jnp.int32(0))
    _sstore(rhi, jn, rows_w)
    nrun = jn + 1

    def _reset_acc():
        for k in range(NSL):
            sl = pl.ds(k * 16, 16)
            z = jnp.zeros((16,), jnp.float32)
            acc[0, sl] = z
            acc[1, sl] = z
            acc[2, sl] = jnp.full((16,), jnp.inf, jnp.float32)
            acc[3, sl] = jnp.full((16,), -jnp.inf, jnp.float32)

    _reset_acc()
    for k in range(NSL):
        sl = pl.ds(k * 16, 16)
        acc[4, sl] = jnp.zeros((16,), jnp.float32)
        acc[5, sl] = jnp.full((16,), -1.0, jnp.float32)
    pltpu.sync_copy(acc, pstats.at[wid * 2 + 1])

    def _flush(j):
        seg = _sload(rseg, j)
        cnt = (_sload(rhi, j) - _sload(rlo, j)).astype(jnp.float32)
        for k in range(NSL):
            acc[4, pl.ds(k * 16, 16)] = jnp.zeros((16,), jnp.float32) + cnt
        is_f = j == 0
        is_l = j == (nrun - 1)

        @pl.when(is_f | is_l)
        def _():
            segf = seg.astype(jnp.float32)
            for k in range(NSL):
                acc[5, pl.ds(k * 16, 16)] = jnp.zeros((16,), jnp.float32) + segf
            slot = wid * 2 + jnp.where(is_f, 0, 1)
            pltpu.sync_copy(acc, pstats.at[slot])

        @pl.when(jnp.logical_not(is_f | is_l))
        def _():
            pltpu.sync_copy(acc.at[pl.ds(0, 5)], dstats.at[seg])

        _reset_acc()

    def _gb(c):
        return jnp.minimum(c * GCH, groups_w - GCH)

    def _dma(c, buf, sem):
        return pltpu.make_async_copy(
            x_hbm.at[pl.ds(gstart + _gb(c), GCH)], buf, sem)

    def _process(ci, j_in, xb):
        rb8 = _gb(ci) * 8
        p_lo = jnp.minimum(ci * GCH * 8, rows_w)
        p_hi = jnp.minimum((ci + 1) * GCH * 8, rows_w)

        def cond(st):
            _, pos = st
            return pos < p_hi

        def body(st):
            jj, pos = st
            rh = _sload(rhi, jj)
            hi = jnp.minimum(rh, p_hi)
            lo_l = pos - rb8
            hi_l = hi - rb8
            gl = (lo_l + 7) // 8
            gh = hi_l // 8
            head_hi = jnp.minimum(gl * 8, hi_l)
            tail_lo = jnp.maximum(gh * 8, head_hi)
            def slice_body(k, _):
                cs = pl.ds(16 * k, 16)
                sl0 = pl.ds(k * 16, 16)
                sl1 = pl.ds(k * 16 + 128, 16)
                t0 = (acc[0, sl0], acc[1, sl0], acc[2, sl0], acc[3, sl0],
                      acc[0, sl1], acc[1, sl1], acc[2, sl1], acc[3, sl1])

                def upd(tt, va, vb):
                    return (tt[0] + va, tt[1] + va * va,
                            jnp.minimum(tt[2], va), jnp.maximum(tt[3], va),
                            tt[4] + vb, tt[5] + vb * vb,
                            jnp.minimum(tt[6], vb), jnp.maximum(tt[7], vb))

                def row_upd(r, tt):
                    return upd(tt, xb[r // 8, 0, r % 8, cs],
                               xb[r // 8, 1, r % 8, cs])

                @plsc.parallel_loop(lo_l, head_hi, carry=t0)
                def t1(r, tt):
                    return row_upd(r, tt)

                @plsc.parallel_loop(gl, gh, unroll=4, carry=t1)
                def t2(gg, tt):
                    for r in range(8):
                        tt = upd(tt, xb[gg, 0, r, cs], xb[gg, 1, r, cs])
                    return tt

                @plsc.parallel_loop(tail_lo, hi_l, carry=t2)
                def t3(r, tt):
                    return row_upd(r, tt)

                acc[0, sl0] = t3[0]
                acc[1, sl0] = t3[1]
                acc[2, sl0] = t3[2]
                acc[3, sl0] = t3[3]
                acc[0, sl1] = t3[4]
                acc[1, sl1] = t3[5]
                acc[2, sl1] = t3[6]
                acc[3, sl1] = t3[7]
                return jnp.int32(0)

            lax.fori_loop(0, NSL // 2, slice_body, jnp.int32(0))
            done = rh <= p_hi

            @pl.when(done)
            def _():
                _flush(jj)

            return jj + done.astype(jnp.int32), hi

        j_out, _ = lax.while_loop(cond, body, (j_in, p_lo))
        return j_out

    _dma(0, xb0, sem0).start()

    def dbl(i, j):
        c0 = 2 * i
        _dma(c0 + 1, xb1, sem1).start()
        _dma(c0, xb0, sem0).wait()
        j = _process(c0, j, xb0)
        _dma(c0 + 2, xb0, sem0).start()
        _dma(c0 + 1, xb1, sem1).wait()
        j = _process(c0 + 1, j, xb1)
        return j

    j = lax.fori_loop(0, (NCH - 1) // 2, dbl, jnp.int32(0))
    _dma(NCH - 1, xb0, sem0).wait()
    _process(NCH - 1, j, xb0)


def _sc_call(x, b2):
    mesh = plsc.VectorSubcoreMesh(core_axis_name="c", subcore_axis_name="s")
    f = pl.kernel(
        _sc_body,
        out_type=[
            jax.ShapeDtypeStruct((G, 5, D), jnp.float32),
            jax.ShapeDtypeStruct((NW * 2, 6, D), jnp.float32),
        ],
        mesh=mesh,
        compiler_params=pltpu.CompilerParams(
            use_tc_tiling_on_sc=False, needs_layout_passes=False),
        scratch_types=[
            pltpu.VMEM((BPAD,), jnp.int32),
            pltpu.VMEM((GCH, 2, 8, 128), jnp.float32),
            pltpu.VMEM((GCH, 2, 8, 128), jnp.float32),
            pltpu.VMEM((6, D), jnp.float32),
            pltpu.VMEM((MAXRUN,), jnp.int32),
            pltpu.VMEM((MAXRUN,), jnp.int32),
            pltpu.VMEM((MAXRUN,), jnp.int32),
            pltpu.SemaphoreType.DMA,
            pltpu.SemaphoreType.DMA,
        ],
    )
    return f(x, b2)


def _tc_body(d_ref, p_ref, w_ref, b_ref, o_ref):
    NP = 2 * NW
    Dst = d_ref[...]
    P = p_ref[...]
    sid = P[:, 5, 0:1]

    key_prev = jnp.concatenate(
        [jnp.full((1, 1), -2.0, jnp.float32), sid[:NP - 1]], axis=0)
    key = jnp.where(sid < 0, key_prev, sid)
    key_next = jnp.concatenate(
        [key[1:], jnp.full((1, 1), jnp.float32(G))], axis=0)

    iota_np = lax.broadcasted_iota(jnp.int32, (NP, G), 1).astype(jnp.float32)
    oh = (key == iota_np).astype(jnp.float32)
    dn = (((0,), (0,)), ((), ()))
    psum = lax.dot_general(oh, P[:, 0, :], dn)
    psq = lax.dot_general(oh, P[:, 1, :], dn)
    pcnt = lax.dot_general(oh, P[:, 4, :], dn)

    mn = P[:, 2, :]
    mx = P[:, 3, :]
    for dstep in (1, 2, 4, 8, 16, 32):
        pad_k = jnp.full((dstep, 1), -3.0, jnp.float32)
        k_s = jnp.concatenate([pad_k, key[:NP - dstep]], axis=0)
        mn_s = jnp.concatenate(
            [jnp.full((dstep, D), jnp.inf, jnp.float32), mn[:NP - dstep]],
            axis=0)
        mx_s = jnp.concatenate(
            [jnp.full((dstep, D), -jnp.inf, jnp.float32), mx[:NP - dstep]],
            axis=0)
        same = k_s == key
        mn = jnp.where(same, jnp.minimum(mn, mn_s), mn)
        mx = jnp.where(same, jnp.maximum(mx, mx_s), mx)
    last = (key != key_next).astype(jnp.float32)
    ohl = oh * last
    pmin = lax.dot_general(ohl, jnp.where(last > 0, mn, 0.0), dn)
    pmax = lax.dot_general(ohl, jnp.where(last > 0, mx, 0.0), dn)
    pmin = jnp.where(pcnt > 0, pmin, jnp.inf)
    pmax = jnp.where(pcnt > 0, pmax, -jnp.inf)

    even = (lax.broadcasted_iota(jnp.int32, (NP, 1), 0) % 2) == 0
    span = (key < iota_np) & (iota_np < key_next) & even
    interior = lax.dot_general(span.astype(jnp.float32),
                               jnp.ones((NP, 1), jnp.float32), dn) > 0.5

    sm = jnp.where(interior, Dst[:, 0, :], psum)
    sq = jnp.where(interior, Dst[:, 1, :], psq)
    mnF = jnp.where(interior, Dst[:, 2, :], pmin)
    mxF = jnp.where(interior, Dst[:, 3, :], pmax)
    cnt = jnp.where(interior, Dst[:, 4, :], pcnt)

    c1 = jnp.maximum(cnt, 1.0)
    mean = sm / c1
    std = jnp.sqrt(jnp.maximum(sq / c1 - mean * mean, 1e-5))

    Wm = w_ref[...]
    o_ref[...] = (sm @ Wm[0:D] + mean @ Wm[D:2 * D] + mnF @ Wm[2 * D:3 * D]
                  + mxF @ Wm[3 * D:4 * D] + std @ Wm[4 * D:5 * D]
                  + b_ref[...])


def _tc_call(dstats, pstats, W, b2):
    return pl.pallas_call(
        _tc_body,
        out_shape=jax.ShapeDtypeStruct((G, OUT), jnp.float32),
    )(dstats, pstats, W, b2)


def kernel(x, batch, W, b):
    bi = batch.astype(jnp.int32)
    b2 = jnp.concatenate(
        [bi, jnp.broadcast_to(bi[N - 1:N], (BPAD,))], axis=0)
    x5 = jnp.transpose(x.reshape(NGRP, 8, 2, 128), (0, 2, 1, 3))
    dstats, pstats = _sc_call(x5, b2)
    return _tc_call(dstats, pstats, W, b.reshape(1, OUT))


# speedup vs baseline: 1.2805x; 1.2805x over previous
import jax
import jax.numpy as jnp
from jax import lax
from jax.experimental import pallas as pl
from jax.experimental.pallas import tpu as pltpu
from jax.experimental.pallas import tpu_sc as plsc

N = 100000
D = 256
G = 512
OUT = 256
NC = 2
NS = 16
NW = NC * NS
NGRP = N // 8
GBASE = NGRP // NW
NBIG = NGRP - GBASE * NW
GCH = 24
NCH = 17
BPAD = 3152
MAXRUN = 544
NSL = D // 16


def _sload(ref, i):
    return ref[pl.ds(i, 16)][0]


def _sstore(ref, i, val):
    lane = lax.broadcasted_iota(jnp.int32, (16,), 0)
    idx = jnp.zeros((16,), jnp.int32) + i
    x = jnp.zeros((16,), ref.dtype) + val
    plsc.store_scatter(ref, [idx], x, mask=lane == 0)


def _sc_body(x_hbm, b2_hbm, dstats, pstats,
             batch_v, xb0, xb1, acc, rlo, rhi, rseg, sem0, sem1):
    cid = lax.axis_index("c")
    sid = lax.axis_index("s")
    wid = sid * NC + cid
    gstart = wid * GBASE + jnp.minimum(wid, NBIG)
    groups_w = GBASE + (wid < NBIG).astype(jnp.int32)
    rows_w = groups_w * 8

    pltpu.sync_copy(b2_hbm.at[pl.ds(gstart * 8, BPAD)], batch_v)

    first = _sload(batch_v, 0)
    _sstore(rlo, 0, jnp.int32(0))
    _sstore(rseg, 0, first)

    def phase_a(i, j):
        r0 = i * 16
        a = batch_v[pl.ds(r0, 16)]
        b = batch_v[pl.ds(r0 + 1, 16)]
        pos = lax.broadcasted_iota(jnp.int32, (16,), 0) + (r0 + 1)
        m = (a != b) & (pos < rows_w)
        nb = plsc.all_reduce_population_count(m)[0]

        @pl.when(nb > 0)
        def _():
            plsc.store_compressed(rhi.at[pl.ds(j, 16)], pos, mask=m)
            plsc.store_compressed(rlo.at[pl.ds(j + 1, 16)], pos, mask=m)
            plsc.store_compressed(rseg.at[pl.ds(j + 1, 16)], b, mask=m)

        return j + nb

    jn = lax.fori_loop(0, (8 * (GBASE + 1) + 14) // 16, phase_a, jnp.int32(0))
    _sstore(rhi, jn, rows_w)
    nrun = jn + 1

    def _reset_acc():
        for k in range(NSL):
            sl = pl.ds(k * 16, 16)
            z = jnp.zeros((16,), jnp.float32)
            acc[0, sl] = z
            acc[1, sl] = z
            acc[2, sl] = jnp.full((16,), jnp.inf, jnp.float32)
            acc[3, sl] = jnp.full((16,), -jnp.inf, jnp.float32)

    _reset_acc()
    for k in range(NSL):
        sl = pl.ds(k * 16, 16)
        acc[4, sl] = jnp.zeros((16,), jnp.float32)
        acc[5, sl] = jnp.full((16,), -1.0, jnp.float32)
    pltpu.sync_copy(acc, pstats.at[wid * 2 + 1])

    def _flush(j):
        seg = _sload(rseg, j)
        cnt = (_sload(rhi, j) - _sload(rlo, j)).astype(jnp.float32)
        for k in range(NSL):
            acc[4, pl.ds(k * 16, 16)] = jnp.zeros((16,), jnp.float32) + cnt
        is_f = j == 0
        is_l = j == (nrun - 1)

        @pl.when(is_f | is_l)
        def _():
            segf = seg.astype(jnp.float32)
            for k in range(NSL):
                acc[5, pl.ds(k * 16, 16)] = jnp.zeros((16,), jnp.float32) + segf
            slot = wid * 2 + jnp.where(is_f, 0, 1)
            pltpu.sync_copy(acc, pstats.at[slot])

        @pl.when(jnp.logical_not(is_f | is_l))
        def _():
            pltpu.sync_copy(acc.at[pl.ds(0, 5)], dstats.at[seg])

        _reset_acc()

    def _gb(c):
        return jnp.minimum(c * GCH, groups_w - GCH)

    def _dma(c, buf, sem):
        return pltpu.make_async_copy(
            x_hbm.at[pl.ds(gstart + _gb(c), GCH)], buf, sem)

    def _process(ci, j_in, xb):
        rb8 = _gb(ci) * 8
        p_lo = jnp.minimum(ci * GCH * 8, rows_w)
        p_hi = jnp.minimum((ci + 1) * GCH * 8, rows_w)

        def cond(st):
            _, pos = st
            return pos < p_hi

        def body(st):
            jj, pos = st
            rh = _sload(rhi, jj)
            hi = jnp.minimum(rh, p_hi)
            lo_l = pos - rb8
            hi_l = hi - rb8
            gl = (lo_l + 7) // 8
            gh = hi_l // 8
            head_hi = jnp.minimum(gl * 8, hi_l)
            tail_lo = jnp.maximum(gh * 8, head_hi)
            def slice_body(k, _):
                cs = pl.ds(16 * k, 16)
                sl0 = pl.ds(k * 16, 16)
                sl1 = pl.ds(k * 16 + 128, 16)
                t0 = (acc[0, sl0], acc[1, sl0], acc[2, sl0], acc[3, sl0],
                      acc[0, sl1], acc[1, sl1], acc[2, sl1], acc[3, sl1])

                def upd(tt, va, vb):
                    return (tt[0] + va, tt[1] + va * va,
                            jnp.minimum(tt[2], va), jnp.maximum(tt[3], va),
                            tt[4] + vb, tt[5] + vb * vb,
                            jnp.minimum(tt[6], vb), jnp.maximum(tt[7], vb))

                def row_upd(r, tt):
                    return upd(tt, xb[r // 8, 0, r % 8, cs],
                               xb[r // 8, 1, r % 8, cs])

                @plsc.parallel_loop(lo_l, head_hi, carry=t0)
                def t1(r, tt):
                    return row_upd(r, tt)

                @plsc.parallel_loop(gl, gh, unroll=2, carry=t1)
                def t2(gg, tt):
                    for r in range(8):
                        tt = upd(tt, xb[gg, 0, r, cs], xb[gg, 1, r, cs])
                    return tt

                @plsc.parallel_loop(tail_lo, hi_l, carry=t2)
                def t3(r, tt):
                    return row_upd(r, tt)

                acc[0, sl0] = t3[0]
                acc[1, sl0] = t3[1]
                acc[2, sl0] = t3[2]
                acc[3, sl0] = t3[3]
                acc[0, sl1] = t3[4]
                acc[1, sl1] = t3[5]
                acc[2, sl1] = t3[6]
                acc[3, sl1] = t3[7]
                return jnp.int32(0)

            lax.fori_loop(0, NSL // 2, slice_body, jnp.int32(0))
            done = rh <= p_hi

            @pl.when(done)
            def _():
                _flush(jj)

            return jj + done.astype(jnp.int32), hi

        j_out, _ = lax.while_loop(cond, body, (j_in, p_lo))
        return j_out

    _dma(0, xb0, sem0).start()

    def dbl(i, j):
        c0 = 2 * i
        _dma(c0 + 1, xb1, sem1).start()
        _dma(c0, xb0, sem0).wait()
        j = _process(c0, j, xb0)
        _dma(c0 + 2, xb0, sem0).start()
        _dma(c0 + 1, xb1, sem1).wait()
        j = _process(c0 + 1, j, xb1)
        return j

    j = lax.fori_loop(0, (NCH - 1) // 2, dbl, jnp.int32(0))
    _dma(NCH - 1, xb0, sem0).wait()
    _process(NCH - 1, j, xb0)


def _sc_call(x, b2):
    mesh = plsc.VectorSubcoreMesh(core_axis_name="c", subcore_axis_name="s")
    f = pl.kernel(
        _sc_body,
        out_type=[
            jax.ShapeDtypeStruct((G, 5, D), jnp.float32),
            jax.ShapeDtypeStruct((NW * 2, 6, D), jnp.float32),
        ],
        mesh=mesh,
        compiler_params=pltpu.CompilerParams(
            use_tc_tiling_on_sc=False, needs_layout_passes=False),
        scratch_types=[
            pltpu.VMEM((BPAD,), jnp.int32),
            pltpu.VMEM((GCH, 2, 8, 128), jnp.float32),
            pltpu.VMEM((GCH, 2, 8, 128), jnp.float32),
            pltpu.VMEM((6, D), jnp.float32),
            pltpu.VMEM((MAXRUN,), jnp.int32),
            pltpu.VMEM((MAXRUN,), jnp.int32),
            pltpu.VMEM((MAXRUN,), jnp.int32),
            pltpu.SemaphoreType.DMA,
            pltpu.SemaphoreType.DMA,
        ],
    )
    return f(x, b2)


def _tc_body(d_ref, p_ref, w_ref, b_ref, o_ref):
    NP = 2 * NW
    Dst = d_ref[...]
    P = p_ref[...]
    sid = P[:, 5, 0:1]

    key_prev = jnp.concatenate(
        [jnp.full((1, 1), -2.0, jnp.float32), sid[:NP - 1]], axis=0)
    key = jnp.where(sid < 0, key_prev, sid)
    key_next = jnp.concatenate(
        [key[1:], jnp.full((1, 1), jnp.float32(G))], axis=0)

    iota_np = lax.broadcasted_iota(jnp.int32, (NP, G), 1).astype(jnp.float32)
    oh = (key == iota_np).astype(jnp.float32)
    dn = (((0,), (0,)), ((), ()))
    psum = lax.dot_general(oh, P[:, 0, :], dn)
    psq = lax.dot_general(oh, P[:, 1, :], dn)
    pcnt = lax.dot_general(oh, P[:, 4, :], dn)

    mn = P[:, 2, :]
    mx = P[:, 3, :]
    for dstep in (1, 2, 4, 8, 16, 32):
        pad_k = jnp.full((dstep, 1), -3.0, jnp.float32)
        k_s = jnp.concatenate([pad_k, key[:NP - dstep]], axis=0)
        mn_s = jnp.concatenate(
            [jnp.full((dstep, D), jnp.inf, jnp.float32), mn[:NP - dstep]],
            axis=0)
        mx_s = jnp.concatenate(
            [jnp.full((dstep, D), -jnp.inf, jnp.float32), mx[:NP - dstep]],
            axis=0)
        same = k_s == key
        mn = jnp.where(same, jnp.minimum(mn, mn_s), mn)
        mx = jnp.where(same, jnp.maximum(mx, mx_s), mx)
    last = (key != key_next).astype(jnp.float32)
    ohl = oh * last
    pmin = lax.dot_general(ohl, jnp.where(last > 0, mn, 0.0), dn)
    pmax = lax.dot_general(ohl, jnp.where(last > 0, mx, 0.0), dn)
    pmin = jnp.where(pcnt > 0, pmin, jnp.inf)
    pmax = jnp.where(pcnt > 0, pmax, -jnp.inf)

    even = (lax.broadcasted_iota(jnp.int32, (NP, 1), 0) % 2) == 0
    span = (key < iota_np) & (iota_np < key_next) & even
    interior = lax.dot_general(span.astype(jnp.float32),
                               jnp.ones((NP, 1), jnp.float32), dn) > 0.5

    sm = jnp.where(interior, Dst[:, 0, :], psum)
    sq = jnp.where(interior, Dst[:, 1, :], psq)
    mnF = jnp.where(interior, Dst[:, 2, :], pmin)
    mxF = jnp.where(interior, Dst[:, 3, :], pmax)
    cnt = jnp.where(interior, Dst[:, 4, :], pcnt)

    c1 = jnp.maximum(cnt, 1.0)
    mean = sm / c1
    std = jnp.sqrt(jnp.maximum(sq / c1 - mean * mean, 1e-5))

    Wm = w_ref[...]
    o_ref[...] = (sm @ Wm[0:D] + mean @ Wm[D:2 * D] + mnF @ Wm[2 * D:3 * D]
                  + mxF @ Wm[3 * D:4 * D] + std @ Wm[4 * D:5 * D]
                  + b_ref[...])


def _tc_call(dstats, pstats, W, b2):
    return pl.pallas_call(
        _tc_body,
        out_shape=jax.ShapeDtypeStruct((G, OUT), jnp.float32),
    )(dstats, pstats, W, b2)


def kernel(x, batch, W, b):
    bi = batch.astype(jnp.int32)
    b2 = jnp.concatenate(
        [bi, jnp.broadcast_to(bi[N - 1:N], (BPAD,))], axis=0)
    x5 = jnp.transpose(x.reshape(NGRP, 8, 2, 128), (0, 2, 1, 3))
    dstats, pstats = _sc_call(x5, b2)
    return _tc_call(dstats, pstats, W, b.reshape(1, OUT))
